# baseline (speedup 1.0000x reference)
import functools

import jax
import jax.numpy as jnp
from jax import lax
from jax.experimental import pallas as pl
from jax.experimental.pallas import tpu as pltpu
from jax.experimental.pallas import tpu_sc as plsc

_N_ATOMS = 10000
_N_PAD = 10240
_CHUNK = 80
_K = 5


def _swish(v):
    return v * jax.nn.sigmoid(v)


def _edge_body(x_ref, rbf_ref, wx_ref, bx_ref, wr_ref, br_ref, out_ref):
    h = jnp.dot(x_ref[...], wx_ref[...], preferred_element_type=jnp.float32)
    h = _swish(h + bx_ref[...])
    r = jnp.dot(rbf_ref[...], wr_ref[...], preferred_element_type=jnp.float32)
    r = _swish(r + br_ref[...])
    out_ref[...] = h * r


def _edge_stage(x, rbf, W_x, b_x, W_rbf, b_rbf, k_slice, block_e):
    E, EMB = x.shape
    RBF = rbf.shape[1]
    blocks_per_slice = E // _K // block_e
    off = k_slice * blocks_per_slice
    return pl.pallas_call(
        _edge_body,
        grid=(blocks_per_slice,),
        in_specs=[
            pl.BlockSpec((block_e, EMB), lambda i: (off + i, 0)),
            pl.BlockSpec((block_e, RBF), lambda i: (off + i, 0)),
            pl.BlockSpec((EMB, EMB), lambda i: (0, 0)),
            pl.BlockSpec((1, EMB), lambda i: (0, 0)),
            pl.BlockSpec((RBF, EMB), lambda i: (0, 0)),
            pl.BlockSpec((1, EMB), lambda i: (0, 0)),
        ],
        out_specs=pl.BlockSpec((block_e, EMB), lambda i: (i, 0)),
        out_shape=jax.ShapeDtypeStruct((E // _K, EMB), jnp.float32),
    )(x, rbf, W_x, b_x.reshape(1, EMB), W_rbf, b_rbf.reshape(1, EMB))


def _seg_sum_sc(conv3, idx4, k_slice):
    n_chunks, _, EMB = conv3.shape
    info = plsc.get_sparse_core_info()
    NC, NS = info.num_cores, info.num_subcores
    nw = NC * NS
    chunks_per_w = n_chunks // nw
    rows_per_s = _N_PAD // NS
    mesh = plsc.VectorSubcoreMesh(core_axis_name="c", subcore_axis_name="s")
    nbuf = 4
    n_groups = chunks_per_w // nbuf
    n_tail = chunks_per_w - n_groups * nbuf

    @functools.partial(
        pl.kernel,
        mesh=mesh,
        out_type=jax.ShapeDtypeStruct((NC, _N_PAD, EMB), jnp.float32),
        scratch_types=[
            pltpu.VMEM((chunks_per_w, _CHUNK), jnp.int32),
            pltpu.VMEM((nbuf, _CHUNK, EMB), jnp.float32),
            pltpu.VMEM_SHARED((_N_PAD, EMB), jnp.float32),
            pltpu.SemaphoreType.DMA((nbuf,)),
            pltpu.SemaphoreType.DMA((nbuf,)),
        ],
    )
    def k(conv_hbm, idx_hbm, out_hbm, idx_v, bufs, acc, semf, sems):
        c = lax.axis_index("c")
        s = lax.axis_index("s")
        w = c * NS + s
        base = w * chunks_per_w
        my_rows = pl.ds(s * rows_per_s, rows_per_s)
        zero16 = jnp.zeros((16,), jnp.float32)

        def zfill(r, carry):
            for c16 in range(EMB // 16):
                bufs[0, r, pl.ds(c16 * 16, 16)] = zero16
            return carry

        lax.fori_loop(0, _CHUNK, zfill, None)
        for r in range(rows_per_s // _CHUNK):
            pltpu.sync_copy(
                bufs.at[0], acc.at[pl.ds(s * rows_per_s + r * _CHUNK, _CHUNK)])
        pltpu.sync_copy(idx_hbm.at[k_slice].at[w], idx_v)
        plsc.subcore_barrier()

        for b in range(nbuf):
            pltpu.async_copy(conv_hbm.at[base + b], bufs.at[b], semf.at[b])

        def body(g, carry):
            j0 = nbuf * g
            for b in range(nbuf):
                pltpu.make_async_copy(
                    conv_hbm.at[base], bufs.at[b], semf.at[b]).wait()
                pltpu.async_copy(
                    bufs.at[b], acc.at[idx_v.at[j0 + b]], sems.at[b],
                    add=True)
            for b in range(nbuf):
                pltpu.make_async_copy(
                    bufs.at[b], acc.at[idx_v.at[0]], sems.at[b]).wait()
                nxt = j0 + nbuf + b

                @pl.when(nxt < chunks_per_w)
                def _():
                    pltpu.async_copy(
                        conv_hbm.at[base + nxt], bufs.at[b], semf.at[b])
            return carry

        lax.fori_loop(0, n_groups, body, None)
        for t in range(n_tail):
            pltpu.make_async_copy(
                conv_hbm.at[base], bufs.at[t], semf.at[t]).wait()
            pltpu.sync_copy(
                bufs.at[t], acc.at[idx_v.at[n_groups * nbuf + t]], add=True)
        plsc.subcore_barrier()
        pltpu.sync_copy(acc.at[my_rows], out_hbm.at[c].at[my_rows])

    return k(conv3, idx4)


def _node_body(*refs):
    p_refs = refs[:_K]
    w1_ref, b1_ref, w2_ref, b2_ref, w3_ref, b3_ref, wf_ref, out_ref = refs[_K:]
    h = p_refs[0][0] + p_refs[0][1]
    for p in p_refs[1:]:
        h = h + p[0] + p[1]
    h = _swish(jnp.dot(h, w1_ref[...], preferred_element_type=jnp.float32)
               + b1_ref[...])
    h = _swish(jnp.dot(h, w2_ref[...], preferred_element_type=jnp.float32)
               + b2_ref[...])
    h = _swish(jnp.dot(h, w3_ref[...], preferred_element_type=jnp.float32)
               + b3_ref[...])
    out_ref[...] = jnp.dot(h, wf_ref[...], preferred_element_type=jnp.float32)


def _node_stage(partials, W1, b1, W2, b2, W3, b3, W_final, n_out, block_n):
    NC, _, EMB = partials[0].shape
    grid = (n_out // block_n,)
    return pl.pallas_call(
        _node_body,
        grid=grid,
        in_specs=[
            pl.BlockSpec((NC, block_n, EMB), lambda i: (0, i, 0))
            for _ in range(_K)
        ] + [
            pl.BlockSpec((EMB, EMB), lambda i: (0, 0)),
            pl.BlockSpec((1, EMB), lambda i: (0, 0)),
            pl.BlockSpec((EMB, EMB), lambda i: (0, 0)),
            pl.BlockSpec((1, EMB), lambda i: (0, 0)),
            pl.BlockSpec((EMB, EMB), lambda i: (0, 0)),
            pl.BlockSpec((1, EMB), lambda i: (0, 0)),
            pl.BlockSpec((EMB, 1), lambda i: (0, 0)),
        ],
        out_specs=pl.BlockSpec((block_n, 1), lambda i: (i, 0)),
        out_shape=jax.ShapeDtypeStruct((n_out, 1), jnp.float32),
    )(*partials, W1, b1.reshape(1, EMB), W2, b2.reshape(1, EMB),
      W3, b3.reshape(1, EMB), W_final)


def kernel(x, rbf, idnb_i, n_atoms, coef_rbf_a, coef_rbf_b, coef_x,
           coef_final, W_x, b_x, W_rbf, b_rbf, W1, b1, W2, b2, W3, b3,
           W_final):
    E, EMB = x.shape
    W_rbf_s = W_rbf * coef_rbf_a[0]
    W1_s = W1 * (coef_x[0] * coef_rbf_b[0])
    W_final_s = W_final * coef_final[0]

    nw = 32
    n_chunks = E // _CHUNK
    cpw = n_chunks // _K // nw
    idx4 = idnb_i.reshape(_K, nw, cpw, _CHUNK)

    partials = []
    for ks in range(_K):
        conv = _edge_stage(x, rbf, W_x, b_x, W_rbf_s, b_rbf, ks,
                           block_e=2560)
        conv3 = conv.reshape(E // _K // _CHUNK, _CHUNK, EMB)
        partials.append(_seg_sum_sc(conv3, idx4, ks))

    return _node_stage(partials, W1_s, b1, W2, b2, W3, b3, W_final_s,
                       n_out=_N_ATOMS, block_n=2000)


# speedup vs baseline: 1.4050x; 1.4050x over previous
import functools

import jax
import jax.numpy as jnp
from jax import lax
from jax.experimental import pallas as pl
from jax.experimental.pallas import tpu as pltpu
from jax.experimental.pallas import tpu_sc as plsc

_N_ATOMS = 10000
_N_PAD = 10240
_CHUNK = 80
_K = 5


def _swish(v):
    return v * jax.nn.sigmoid(v)


def _edge_body(x_ref, rbft_ref, wx_ref, bx_ref, wr_ref, br_ref, out_ref):
    h = jnp.dot(x_ref[...], wx_ref[...], preferred_element_type=jnp.float32)
    h = _swish(h + bx_ref[...])
    r = lax.dot_general(rbft_ref[...], wr_ref[...],
                        dimension_numbers=(((0,), (0,)), ((), ())),
                        preferred_element_type=jnp.float32)
    r = _swish(r + br_ref[...])
    out_ref[...] = h * r


def _edge_stage(x, rbf_t, W_x, b_x, W_rbf, b_rbf, k_slice, block_e):
    E, EMB = x.shape
    RBF = rbf_t.shape[0]
    blocks_per_slice = E // _K // block_e
    off = k_slice * blocks_per_slice
    return pl.pallas_call(
        _edge_body,
        grid=(blocks_per_slice,),
        in_specs=[
            pl.BlockSpec((block_e, EMB), lambda i: (off + i, 0)),
            pl.BlockSpec((RBF, block_e), lambda i: (0, off + i)),
            pl.BlockSpec((EMB, EMB), lambda i: (0, 0)),
            pl.BlockSpec((1, EMB), lambda i: (0, 0)),
            pl.BlockSpec((RBF, EMB), lambda i: (0, 0)),
            pl.BlockSpec((1, EMB), lambda i: (0, 0)),
        ],
        out_specs=pl.BlockSpec((block_e, EMB), lambda i: (i, 0)),
        out_shape=jax.ShapeDtypeStruct((E // _K, EMB), jnp.float32),
    )(x, rbf_t, W_x, b_x.reshape(1, EMB), W_rbf, b_rbf.reshape(1, EMB))


def _seg_sum_sc(conv3, idx4, k_slice):
    n_chunks, _, EMB = conv3.shape
    info = plsc.get_sparse_core_info()
    NC, NS = info.num_cores, info.num_subcores
    nw = NC * NS
    chunks_per_w = n_chunks // nw
    rows_per_s = _N_PAD // NS
    mesh = plsc.VectorSubcoreMesh(core_axis_name="c", subcore_axis_name="s")
    nbuf = 4
    n_groups = chunks_per_w // nbuf
    n_tail = chunks_per_w - n_groups * nbuf

    @functools.partial(
        pl.kernel,
        mesh=mesh,
        out_type=jax.ShapeDtypeStruct((NC, _N_PAD, EMB), jnp.float32),
        scratch_types=[
            pltpu.VMEM((chunks_per_w, _CHUNK), jnp.int32),
            pltpu.VMEM((nbuf, _CHUNK, EMB), jnp.float32),
            pltpu.VMEM_SHARED((_N_PAD, EMB), jnp.float32),
            pltpu.SemaphoreType.DMA((nbuf,)),
            pltpu.SemaphoreType.DMA((nbuf,)),
        ],
    )
    def k(conv_hbm, idx_hbm, out_hbm, idx_v, bufs, acc, semf, sems):
        c = lax.axis_index("c")
        s = lax.axis_index("s")
        w = c * NS + s
        base = w * chunks_per_w
        my_rows = pl.ds(s * rows_per_s, rows_per_s)
        zero16 = jnp.zeros((16,), jnp.float32)

        def zfill(r, carry):
            for c16 in range(EMB // 16):
                bufs[0, r, pl.ds(c16 * 16, 16)] = zero16
            return carry

        lax.fori_loop(0, _CHUNK, zfill, None)
        for r in range(rows_per_s // _CHUNK):
            pltpu.sync_copy(
                bufs.at[0], acc.at[pl.ds(s * rows_per_s + r * _CHUNK, _CHUNK)])
        pltpu.sync_copy(idx_hbm.at[k_slice].at[w], idx_v)
        plsc.subcore_barrier()

        for b in range(nbuf):
            pltpu.async_copy(conv_hbm.at[base + b], bufs.at[b], semf.at[b])

        def body(g, carry):
            j0 = nbuf * g
            for b in range(nbuf):
                pltpu.make_async_copy(
                    conv_hbm.at[base], bufs.at[b], semf.at[b]).wait()
                pltpu.async_copy(
                    bufs.at[b], acc.at[idx_v.at[j0 + b]], sems.at[b],
                    add=True)
            for b in range(nbuf):
                pltpu.make_async_copy(
                    bufs.at[b], acc.at[idx_v.at[0]], sems.at[b]).wait()
                nxt = j0 + nbuf + b

                @pl.when(nxt < chunks_per_w)
                def _():
                    pltpu.async_copy(
                        conv_hbm.at[base + nxt], bufs.at[b], semf.at[b])
            return carry

        lax.fori_loop(0, n_groups, body, None)
        for t in range(n_tail):
            pltpu.make_async_copy(
                conv_hbm.at[base], bufs.at[t], semf.at[t]).wait()
            pltpu.sync_copy(
                bufs.at[t], acc.at[idx_v.at[n_groups * nbuf + t]], add=True)
        plsc.subcore_barrier()
        pltpu.sync_copy(acc.at[my_rows], out_hbm.at[c].at[my_rows])

    return k(conv3, idx4)


def _node_body(*refs):
    p_refs = refs[:_K]
    w1_ref, b1_ref, w2_ref, b2_ref, w3_ref, b3_ref, wf_ref, out_ref = refs[_K:]
    h = p_refs[0][0] + p_refs[0][1]
    for p in p_refs[1:]:
        h = h + p[0] + p[1]
    h = _swish(jnp.dot(h, w1_ref[...], preferred_element_type=jnp.float32)
               + b1_ref[...])
    h = _swish(jnp.dot(h, w2_ref[...], preferred_element_type=jnp.float32)
               + b2_ref[...])
    h = _swish(jnp.dot(h, w3_ref[...], preferred_element_type=jnp.float32)
               + b3_ref[...])
    out_ref[...] = jnp.dot(h, wf_ref[...], preferred_element_type=jnp.float32)


def _node_stage(partials, W1, b1, W2, b2, W3, b3, W_final, n_out, block_n):
    NC, _, EMB = partials[0].shape
    grid = (n_out // block_n,)
    return pl.pallas_call(
        _node_body,
        grid=grid,
        in_specs=[
            pl.BlockSpec((NC, block_n, EMB), lambda i: (0, i, 0))
            for _ in range(_K)
        ] + [
            pl.BlockSpec((EMB, EMB), lambda i: (0, 0)),
            pl.BlockSpec((1, EMB), lambda i: (0, 0)),
            pl.BlockSpec((EMB, EMB), lambda i: (0, 0)),
            pl.BlockSpec((1, EMB), lambda i: (0, 0)),
            pl.BlockSpec((EMB, EMB), lambda i: (0, 0)),
            pl.BlockSpec((1, EMB), lambda i: (0, 0)),
            pl.BlockSpec((EMB, 1), lambda i: (0, 0)),
        ],
        out_specs=pl.BlockSpec((block_n, 1), lambda i: (i, 0)),
        out_shape=jax.ShapeDtypeStruct((n_out, 1), jnp.float32),
    )(*partials, W1, b1.reshape(1, EMB), W2, b2.reshape(1, EMB),
      W3, b3.reshape(1, EMB), W_final)


def kernel(x, rbf, idnb_i, n_atoms, coef_rbf_a, coef_rbf_b, coef_x,
           coef_final, W_x, b_x, W_rbf, b_rbf, W1, b1, W2, b2, W3, b3,
           W_final):
    E, EMB = x.shape
    W_rbf_s = W_rbf * coef_rbf_a[0]
    W1_s = W1 * (coef_x[0] * coef_rbf_b[0])
    W_final_s = W_final * coef_final[0]

    nw = 32
    n_chunks = E // _CHUNK
    cpw = n_chunks // _K // nw
    idx4 = idnb_i.reshape(_K, nw, cpw, _CHUNK)

    rbf_t = rbf.T
    partials = []
    for ks in range(_K):
        conv = _edge_stage(x, rbf_t, W_x, b_x, W_rbf_s, b_rbf, ks,
                           block_e=2560)
        conv3 = conv.reshape(E // _K // _CHUNK, _CHUNK, EMB)
        partials.append(_seg_sum_sc(conv3, idx4, ks))

    return _node_stage(partials, W1_s, b1, W2, b2, W3, b3, W_final_s,
                       n_out=_N_ATOMS, block_n=2000)
